# baseline (speedup 1.0000x reference)
import functools

import jax
import jax.numpy as jnp
from jax import lax
from jax.experimental import pallas as pl
from jax.experimental.pallas import tpu as pltpu
from jax.experimental.pallas import tpu_sc as plsc

FDIM = 24
KNN = 16
_INF = float("inf")


def _lrelu(v):
    return jnp.where(v >= 0, v, 0.2 * v)


def _dot(a, b):
    return jnp.dot(a, b, preferred_element_type=jnp.float32)


@functools.lru_cache(maxsize=None)
def _make_knn(B, N, Ns, sr, dil, keep, R):
    n_iters = (keep - 1) * dil + 1

    def body(pos_ref, poss_ref, out_ref, dist_s):
        b = pl.program_id(0)
        p = pos_ref[0]
        q = poss_ref[0]
        p2 = jnp.sum(p * p, axis=1, keepdims=True)
        q2 = jnp.sum(q * q, axis=1)[None, :]
        cross = lax.dot_general(p, q, (((1,), (1,)), ((), ())),
                                preferred_element_type=jnp.float32)
        d2 = p2 + q2 - 2.0 * cross
        dist_s[...] = jnp.sqrt(jnp.maximum(d2, 1e-12))
        iota = lax.broadcasted_iota(jnp.int32, (R, Ns), 1)
        base = b * N

        def step(it, carry):
            mp, sp = carry
            cdv = dist_s[...]
            live = (cdv > mp) | ((cdv == mp) & (iota > sp))
            m = jnp.min(jnp.where(live, cdv, _INF), axis=1, keepdims=True)
            sel = jnp.min(jnp.where(live & (cdv == m), iota, Ns), axis=1,
                          keepdims=True)

            @pl.when(it % dil == 0)
            def _():
                k = it // dil
                out_ref[0, pl.ds(k, 1), :] = (base + sel[:, 0] * sr)[None, :]

            return m, sel

        lax.fori_loop(0, n_iters, step,
                      (jnp.full((R, 1), -_INF, jnp.float32),
                       jnp.full((R, 1), -1, jnp.int32)))

    return pl.pallas_call(
        body,
        grid=(B, N // R),
        in_specs=[
            pl.BlockSpec((1, R, 3), lambda b, j: (b, j, 0)),
            pl.BlockSpec((1, Ns, 3), lambda b, j: (b, 0, 0)),
        ],
        out_specs=pl.BlockSpec((1, keep, R), lambda b, j: (b, 0, j)),
        out_shape=jax.ShapeDtypeStruct((B, keep, N), jnp.int32),
        scratch_shapes=[pltpu.VMEM((R, Ns), jnp.float32)],
    )


@functools.lru_cache(maxsize=None)
def _make_gather(M, C):
    info = plsc.get_sparse_core_info()
    NW = info.num_cores * info.num_subcores
    CH = 128
    per_w = M // NW
    n_ch = per_w // CH
    mesh = plsc.VectorSubcoreMesh(core_axis_name="c", subcore_axis_name="s")

    @functools.partial(
        pl.kernel,
        out_type=jax.ShapeDtypeStruct((M, C), jnp.float32),
        mesh=mesh,
        scratch_types=[
            pltpu.VMEM((CH,), jnp.int32),
            pltpu.VMEM((CH, C), jnp.float32),
            pltpu.SemaphoreType.DMA,
        ],
    )
    def gk(table_hbm, idx_hbm, out_hbm, idx_v, rows_v, sem):
        wid = lax.axis_index("s") * info.num_cores + lax.axis_index("c")
        base = wid * per_w

        def step(i, _):
            off = base + i * CH
            pltpu.sync_copy(idx_hbm.at[pl.ds(off, CH)], idx_v)
            pltpu.async_copy(table_hbm.at[idx_v], rows_v, sem).wait()
            pltpu.sync_copy(rows_v, out_hbm.at[pl.ds(off, CH)])
            return 0

        lax.fori_loop(0, n_ch, step, 0)

    return gk


def _gather_rows(table, idx_flat):
    return _make_gather(idx_flat.shape[0], table.shape[1])(table, idx_flat)


@functools.lru_cache(maxsize=None)
def _make_stn_pool(B, N, R):
    def body(x_ref, w1, b1, w2, b2, w3, b3, out_ref):
        j = pl.program_id(1)
        h = jnp.maximum(_dot(x_ref[0], w1[...]) + b1[...], 0)
        h = jnp.maximum(_dot(h, w2[...]) + b2[...], 0)
        h = jnp.maximum(_dot(h, w3[...]) + b3[...], 0)
        m = jnp.max(h, axis=0, keepdims=True)[None]

        @pl.when(j == 0)
        def _():
            out_ref[...] = m

        @pl.when(j > 0)
        def _():
            out_ref[...] = jnp.maximum(out_ref[...], m)

    full = lambda *s: pl.BlockSpec(s, lambda b, j: tuple(0 for _ in s))
    return pl.pallas_call(
        body,
        grid=(B, N // R),
        in_specs=[
            pl.BlockSpec((1, R, FDIM), lambda b, j: (b, j, 0)),
            full(FDIM, 64), full(1, 64),
            full(64, 128), full(1, 128),
            full(128, 1024), full(1, 1024),
        ],
        out_specs=pl.BlockSpec((1, 1, 1024), lambda b, j: (b, 0, 0)),
        out_shape=jax.ShapeDtypeStruct((B, 1, 1024), jnp.float32),
    )


@functools.lru_cache(maxsize=None)
def _make_stn_head(B):
    def body(p_ref, w1, b1, w2, b2, w3, b3, out_ref):
        h = jnp.maximum(_dot(p_ref[...], w1[...]) + b1[...], 0)
        h = jnp.maximum(_dot(h, w2[...]) + b2[...], 0)
        out_ref[...] = _dot(h, w3[...]) + b3[...]

    return pl.pallas_call(
        body,
        out_shape=jax.ShapeDtypeStruct((B, FDIM * FDIM), jnp.float32),
    )


@functools.lru_cache(maxsize=None)
def _make_apply_trans(B, N, R):
    def body(x_ref, t_ref, out_ref):
        xt = _dot(x_ref[0], t_ref[0])
        out_ref[0] = jnp.concatenate(
            [xt, jnp.zeros((R, 128 - FDIM), jnp.float32)], axis=1)

    return pl.pallas_call(
        body,
        grid=(B, N // R),
        in_specs=[
            pl.BlockSpec((1, R, FDIM), lambda b, j: (b, j, 0)),
            pl.BlockSpec((1, FDIM, FDIM), lambda b, j: (b, 0, 0)),
        ],
        out_specs=pl.BlockSpec((1, R, 128), lambda b, j: (b, j, 0)),
        out_shape=jax.ShapeDtypeStruct((B, N, 128), jnp.float32),
    )


@functools.lru_cache(maxsize=None)
def _make_edge(B, N, K, P, Cx, Cp, C1, C2):
    def body(x_ref, g_ref, w1, b1, w2, b2, out_ref):
        xc = x_ref[0][:, :Cx]
        xcb = jnp.broadcast_to(xc[:, None, :], (P, K, Cx)).reshape(P * K, Cx)
        gn = g_ref[0].reshape(P * K, Cp)[:, :Cx]
        e = jnp.concatenate([xcb, gn - xcb], axis=1)
        h = _lrelu(_dot(e, w1[...]) + b1[...])
        h = _lrelu(_dot(h, w2[...]) + b2[...])
        out_ref[0] = jnp.max(h.reshape(P, K, C2), axis=1)

    return pl.pallas_call(
        body,
        grid=(B, N // P),
        in_specs=[
            pl.BlockSpec((1, P, Cp), lambda b, j: (b, j, 0)),
            pl.BlockSpec((1, P, K, Cp), lambda b, j: (b, j, 0, 0)),
            pl.BlockSpec((2 * Cx, C1), lambda b, j: (0, 0)),
            pl.BlockSpec((1, C1), lambda b, j: (0, 0)),
            pl.BlockSpec((C1, C2), lambda b, j: (0, 0)),
            pl.BlockSpec((1, C2), lambda b, j: (0, 0)),
        ],
        out_specs=pl.BlockSpec((1, P, C2), lambda b, j: (b, j, 0)),
        out_shape=jax.ShapeDtypeStruct((B, N, C2), jnp.float32),
    )


@functools.lru_cache(maxsize=None)
def _make_tail(B, N, P, NC):
    def body(x_ref, attw, attb, r1w1, r1b1, r1w2, r1b2,
             r2w1, r2b1, r2w2, r2b2, r2wr, r2br, ow, ob, out_ref):
        xv = x_ref[0]
        logit = jnp.sum(xv * attw[...], axis=1, keepdims=True) + attb[...]
        xv = xv * (1.0 / (1.0 + jnp.exp(-logit)))
        h = _lrelu(_dot(xv, r1w1[...]) + r1b1[...])
        h = _lrelu(_dot(h, r1w2[...]) + r1b2[...])
        xv = h + xv
        h = _lrelu(_dot(xv, r2w1[...]) + r2b1[...])
        h = _lrelu(_dot(h, r2w2[...]) + r2b2[...])
        xv = h + _dot(xv, r2wr[...]) + r2br[...]
        out_ref[0] = _dot(xv, ow[...]) + ob[...]

    full = lambda *s: pl.BlockSpec(s, lambda b, j: tuple(0 for _ in s))
    return pl.pallas_call(
        body,
        grid=(B, N // P),
        in_specs=[
            pl.BlockSpec((1, P, 512), lambda b, j: (b, j, 0)),
            full(1, 512), full(1, 1),
            full(512, 512), full(1, 512), full(512, 512), full(1, 512),
            full(512, 256), full(1, 256), full(256, 256), full(1, 256),
            full(512, 256), full(1, 256),
            full(256, NC), full(1, NC),
        ],
        out_specs=pl.BlockSpec((1, P, NC), lambda b, j: (b, j, 0)),
        out_shape=jax.ShapeDtypeStruct((B, N, NC), jnp.float32),
    )


def _rb(b):
    return b.reshape(1, -1)


def kernel(x, pos, params):
    B, N, D = x.shape
    K = KNN
    R = 256
    P = 256

    idx_local = _make_knn(B, N, N, 1, 1, K, 512)(pos, pos)
    sample_idx = []
    for sr in (4, 8, 16):
        pos_s = pos[:, ::sr, :]
        Ns = pos_s.shape[1]
        sample_idx.append(
            _make_knn(B, N, Ns, sr, 8, K, 2048)(pos, pos_s))

    def flat_idx(ix):
        return ix.transpose(0, 2, 1).reshape(-1)

    ps = params["stn"]
    pooled = _make_stn_pool(B, N, R)(
        x, ps["conv1"]["W"], _rb(ps["conv1"]["b"]),
        ps["conv2"]["W"], _rb(ps["conv2"]["b"]),
        ps["conv3"]["W"], _rb(ps["conv3"]["b"]))
    t = _make_stn_head(B)(
        pooled.reshape(B, 1024), ps["fc1"]["W"], _rb(ps["fc1"]["b"]),
        ps["fc2"]["W"], _rb(ps["fc2"]["b"]),
        ps["fc3"]["W"], _rb(ps["fc3"]["b"]))
    trans = t.reshape(B, FDIM, FDIM)

    pe = params["e_local"]
    xtp = _make_apply_trans(B, N, R)(x, trans)
    g = _gather_rows(xtp.reshape(B * N, 128), flat_idx(idx_local))
    xcur = _make_edge(B, N, K, P, FDIM, 128, 128, 256)(
        xtp, g.reshape(B, N, K, 128),
        pe["l1"]["W"], _rb(pe["l1"]["b"]), pe["l2"]["W"], _rb(pe["l2"]["b"]))

    for i, name in enumerate(("e0", "e1", "e2")):
        pe = params[name]
        Cx = pe["l1"]["W"].shape[0] // 2
        C1 = pe["l1"]["W"].shape[1]
        C2 = pe["l2"]["W"].shape[1]
        g = _gather_rows(xcur.reshape(B * N, Cx), flat_idx(sample_idx[i]))
        xcur = _make_edge(B, N, K, P, Cx, Cx, C1, C2)(
            xcur, g.reshape(B, N, K, Cx),
            pe["l1"]["W"], _rb(pe["l1"]["b"]), pe["l2"]["W"], _rb(pe["l2"]["b"]))

    pa, p1, p2, po = (params["attention"], params["res1"], params["res2"],
                      params["out"])
    NC = po["W"].shape[1]
    out = _make_tail(B, N, P, NC)(
        xcur,
        pa["att"]["W"].reshape(1, 512), pa["att"]["b"].reshape(1, 1),
        p1["l1"]["W"], _rb(p1["l1"]["b"]), p1["l2"]["W"], _rb(p1["l2"]["b"]),
        p2["l1"]["W"], _rb(p2["l1"]["b"]), p2["l2"]["W"], _rb(p2["l2"]["b"]),
        p2["rescale"]["W"], _rb(p2["rescale"]["b"]),
        po["W"], _rb(po["b"]))
    return out


# speedup vs baseline: 1.4566x; 1.4566x over previous
import functools

import jax
import jax.numpy as jnp
from jax import lax
from jax.experimental import pallas as pl
from jax.experimental.pallas import tpu as pltpu
from jax.experimental.pallas import tpu_sc as plsc

FDIM = 24
KNN = 16
_INF = float("inf")


def _lrelu(v):
    return jnp.where(v >= 0, v, 0.2 * v)


def _dot(a, b):
    return jnp.dot(a, b, preferred_element_type=jnp.float32)


@functools.lru_cache(maxsize=None)
def _make_knn(B, N, Ns, sr, dil, keep, R):
    n_iters = (keep - 1) * dil + 1

    def body(pos_ref, poss_ref, out_ref, dist_s):
        b = pl.program_id(0)
        p = pos_ref[0]
        q = poss_ref[0]
        p2 = jnp.sum(p * p, axis=1, keepdims=True)
        q2 = jnp.sum(q * q, axis=1)[None, :]
        cross = lax.dot_general(p, q, (((1,), (1,)), ((), ())),
                                preferred_element_type=jnp.float32)
        d2 = p2 + q2 - 2.0 * cross
        dist_s[...] = jnp.sqrt(jnp.maximum(d2, 1e-12))
        iota = lax.broadcasted_iota(jnp.int32, (R, Ns), 1)
        base = b * N

        def step(it, _):
            cdv = dist_s[...]
            m = jnp.min(cdv, axis=1, keepdims=True)
            sel = jnp.min(jnp.where(cdv == m, iota, Ns), axis=1,
                          keepdims=True)

            @pl.when(it % dil == 0)
            def _():
                k = it // dil
                out_ref[0, pl.ds(k, 1), :] = (base + sel[:, 0] * sr)[None, :]

            dist_s[...] = jnp.where(iota == sel, _INF, cdv)
            return 0

        lax.fori_loop(0, n_iters, step, 0)

    return pl.pallas_call(
        body,
        grid=(B, N // R),
        in_specs=[
            pl.BlockSpec((1, R, 3), lambda b, j: (b, j, 0)),
            pl.BlockSpec((1, Ns, 3), lambda b, j: (b, 0, 0)),
        ],
        out_specs=pl.BlockSpec((1, keep, R), lambda b, j: (b, 0, j)),
        out_shape=jax.ShapeDtypeStruct((B, keep, N), jnp.int32),
        scratch_shapes=[pltpu.VMEM((R, Ns), jnp.float32)],
    )


@functools.lru_cache(maxsize=None)
def _make_gather(M, C):
    info = plsc.get_sparse_core_info()
    NW = info.num_cores * info.num_subcores
    CH = 128
    per_w = M // NW
    n_ch = per_w // CH
    mesh = plsc.VectorSubcoreMesh(core_axis_name="c", subcore_axis_name="s")

    @functools.partial(
        pl.kernel,
        out_type=jax.ShapeDtypeStruct((M, C), jnp.float32),
        mesh=mesh,
        scratch_types=[
            pltpu.VMEM((CH,), jnp.int32),
            pltpu.VMEM((CH, C), jnp.float32),
            pltpu.SemaphoreType.DMA,
        ],
    )
    def gk(table_hbm, idx_hbm, out_hbm, idx_v, rows_v, sem):
        wid = lax.axis_index("s") * info.num_cores + lax.axis_index("c")
        base = wid * per_w

        def step(i, _):
            off = base + i * CH
            pltpu.sync_copy(idx_hbm.at[pl.ds(off, CH)], idx_v)
            pltpu.async_copy(table_hbm.at[idx_v], rows_v, sem).wait()
            pltpu.sync_copy(rows_v, out_hbm.at[pl.ds(off, CH)])
            return 0

        lax.fori_loop(0, n_ch, step, 0)

    return gk


def _gather_rows(table, idx_flat):
    return _make_gather(idx_flat.shape[0], table.shape[1])(table, idx_flat)


@functools.lru_cache(maxsize=None)
def _make_stn_pool(B, N, R):
    def body(x_ref, w1, b1, w2, b2, w3, b3, out_ref):
        j = pl.program_id(1)
        h = jnp.maximum(_dot(x_ref[0], w1[...]) + b1[...], 0)
        h = jnp.maximum(_dot(h, w2[...]) + b2[...], 0)
        h = jnp.maximum(_dot(h, w3[...]) + b3[...], 0)
        m = jnp.max(h, axis=0, keepdims=True)[None]

        @pl.when(j == 0)
        def _():
            out_ref[...] = m

        @pl.when(j > 0)
        def _():
            out_ref[...] = jnp.maximum(out_ref[...], m)

    full = lambda *s: pl.BlockSpec(s, lambda b, j: tuple(0 for _ in s))
    return pl.pallas_call(
        body,
        grid=(B, N // R),
        in_specs=[
            pl.BlockSpec((1, R, FDIM), lambda b, j: (b, j, 0)),
            full(FDIM, 64), full(1, 64),
            full(64, 128), full(1, 128),
            full(128, 1024), full(1, 1024),
        ],
        out_specs=pl.BlockSpec((1, 1, 1024), lambda b, j: (b, 0, 0)),
        out_shape=jax.ShapeDtypeStruct((B, 1, 1024), jnp.float32),
    )


@functools.lru_cache(maxsize=None)
def _make_stn_head(B):
    def body(p_ref, w1, b1, w2, b2, w3, b3, out_ref):
        h = jnp.maximum(_dot(p_ref[...], w1[...]) + b1[...], 0)
        h = jnp.maximum(_dot(h, w2[...]) + b2[...], 0)
        out_ref[...] = _dot(h, w3[...]) + b3[...]

    return pl.pallas_call(
        body,
        out_shape=jax.ShapeDtypeStruct((B, FDIM * FDIM), jnp.float32),
    )


@functools.lru_cache(maxsize=None)
def _make_apply_trans(B, N, R):
    def body(x_ref, t_ref, out_ref):
        xt = _dot(x_ref[0], t_ref[0])
        out_ref[0] = jnp.concatenate(
            [xt, jnp.zeros((R, 128 - FDIM), jnp.float32)], axis=1)

    return pl.pallas_call(
        body,
        grid=(B, N // R),
        in_specs=[
            pl.BlockSpec((1, R, FDIM), lambda b, j: (b, j, 0)),
            pl.BlockSpec((1, FDIM, FDIM), lambda b, j: (b, 0, 0)),
        ],
        out_specs=pl.BlockSpec((1, R, 128), lambda b, j: (b, j, 0)),
        out_shape=jax.ShapeDtypeStruct((B, N, 128), jnp.float32),
    )


@functools.lru_cache(maxsize=None)
def _make_edge(B, N, K, P, Cx, Cp, C1, C2):
    def body(x_ref, g_ref, w1, b1, w2, b2, out_ref):
        xc = x_ref[0][:, :Cx]
        xcb = jnp.broadcast_to(xc[:, None, :], (P, K, Cx)).reshape(P * K, Cx)
        gn = g_ref[0].reshape(P * K, Cp)[:, :Cx]
        e = jnp.concatenate([xcb, gn - xcb], axis=1)
        h = _lrelu(_dot(e, w1[...]) + b1[...])
        h = _lrelu(_dot(h, w2[...]) + b2[...])
        out_ref[0] = jnp.max(h.reshape(P, K, C2), axis=1)

    return pl.pallas_call(
        body,
        grid=(B, N // P),
        in_specs=[
            pl.BlockSpec((1, P, Cp), lambda b, j: (b, j, 0)),
            pl.BlockSpec((1, P, K, Cp), lambda b, j: (b, j, 0, 0)),
            pl.BlockSpec((2 * Cx, C1), lambda b, j: (0, 0)),
            pl.BlockSpec((1, C1), lambda b, j: (0, 0)),
            pl.BlockSpec((C1, C2), lambda b, j: (0, 0)),
            pl.BlockSpec((1, C2), lambda b, j: (0, 0)),
        ],
        out_specs=pl.BlockSpec((1, P, C2), lambda b, j: (b, j, 0)),
        out_shape=jax.ShapeDtypeStruct((B, N, C2), jnp.float32),
    )


@functools.lru_cache(maxsize=None)
def _make_tail(B, N, P, NC):
    def body(x_ref, attw, attb, r1w1, r1b1, r1w2, r1b2,
             r2w1, r2b1, r2w2, r2b2, r2wr, r2br, ow, ob, out_ref):
        xv = x_ref[0]
        logit = jnp.sum(xv * attw[...], axis=1, keepdims=True) + attb[...]
        xv = xv * (1.0 / (1.0 + jnp.exp(-logit)))
        h = _lrelu(_dot(xv, r1w1[...]) + r1b1[...])
        h = _lrelu(_dot(h, r1w2[...]) + r1b2[...])
        xv = h + xv
        h = _lrelu(_dot(xv, r2w1[...]) + r2b1[...])
        h = _lrelu(_dot(h, r2w2[...]) + r2b2[...])
        xv = h + _dot(xv, r2wr[...]) + r2br[...]
        out_ref[0] = _dot(xv, ow[...]) + ob[...]

    full = lambda *s: pl.BlockSpec(s, lambda b, j: tuple(0 for _ in s))
    return pl.pallas_call(
        body,
        grid=(B, N // P),
        in_specs=[
            pl.BlockSpec((1, P, 512), lambda b, j: (b, j, 0)),
            full(1, 512), full(1, 1),
            full(512, 512), full(1, 512), full(512, 512), full(1, 512),
            full(512, 256), full(1, 256), full(256, 256), full(1, 256),
            full(512, 256), full(1, 256),
            full(256, NC), full(1, NC),
        ],
        out_specs=pl.BlockSpec((1, P, NC), lambda b, j: (b, j, 0)),
        out_shape=jax.ShapeDtypeStruct((B, N, NC), jnp.float32),
    )


def _rb(b):
    return b.reshape(1, -1)


def kernel(x, pos, params):
    B, N, D = x.shape
    K = KNN
    R = 256
    P = 256

    idx_local = _make_knn(B, N, N, 1, 1, K, 512)(pos, pos)
    sample_idx = []
    for sr in (4, 8, 16):
        pos_s = pos[:, ::sr, :]
        Ns = pos_s.shape[1]
        sample_idx.append(
            _make_knn(B, N, Ns, sr, 8, K, 2048)(pos, pos_s))

    def flat_idx(ix):
        return ix.transpose(0, 2, 1).reshape(-1)

    ps = params["stn"]
    pooled = _make_stn_pool(B, N, R)(
        x, ps["conv1"]["W"], _rb(ps["conv1"]["b"]),
        ps["conv2"]["W"], _rb(ps["conv2"]["b"]),
        ps["conv3"]["W"], _rb(ps["conv3"]["b"]))
    t = _make_stn_head(B)(
        pooled.reshape(B, 1024), ps["fc1"]["W"], _rb(ps["fc1"]["b"]),
        ps["fc2"]["W"], _rb(ps["fc2"]["b"]),
        ps["fc3"]["W"], _rb(ps["fc3"]["b"]))
    trans = t.reshape(B, FDIM, FDIM)

    pe = params["e_local"]
    xtp = _make_apply_trans(B, N, R)(x, trans)
    g = _gather_rows(xtp.reshape(B * N, 128), flat_idx(idx_local))
    xcur = _make_edge(B, N, K, P, FDIM, 128, 128, 256)(
        xtp, g.reshape(B, N, K, 128),
        pe["l1"]["W"], _rb(pe["l1"]["b"]), pe["l2"]["W"], _rb(pe["l2"]["b"]))

    for i, name in enumerate(("e0", "e1", "e2")):
        pe = params[name]
        Cx = pe["l1"]["W"].shape[0] // 2
        C1 = pe["l1"]["W"].shape[1]
        C2 = pe["l2"]["W"].shape[1]
        g = _gather_rows(xcur.reshape(B * N, Cx), flat_idx(sample_idx[i]))
        xcur = _make_edge(B, N, K, P, Cx, Cx, C1, C2)(
            xcur, g.reshape(B, N, K, Cx),
            pe["l1"]["W"], _rb(pe["l1"]["b"]), pe["l2"]["W"], _rb(pe["l2"]["b"]))

    pa, p1, p2, po = (params["attention"], params["res1"], params["res2"],
                      params["out"])
    NC = po["W"].shape[1]
    out = _make_tail(B, N, P, NC)(
        xcur,
        pa["att"]["W"].reshape(1, 512), pa["att"]["b"].reshape(1, 1),
        p1["l1"]["W"], _rb(p1["l1"]["b"]), p1["l2"]["W"], _rb(p1["l2"]["b"]),
        p2["l1"]["W"], _rb(p2["l1"]["b"]), p2["l2"]["W"], _rb(p2["l2"]["b"]),
        p2["rescale"]["W"], _rb(p2["rescale"]["b"]),
        po["W"], _rb(po["b"]))
    return out


# speedup vs baseline: 1.4989x; 1.0290x over previous
import functools

import jax
import jax.numpy as jnp
from jax import lax
from jax.experimental import pallas as pl
from jax.experimental.pallas import tpu as pltpu
from jax.experimental.pallas import tpu_sc as plsc

FDIM = 24
KNN = 16
_INF = float("inf")


def _lrelu(v):
    return jnp.where(v >= 0, v, 0.2 * v)


def _dot(a, b):
    return jnp.dot(a, b, preferred_element_type=jnp.float32)


@functools.lru_cache(maxsize=None)
def _make_knn(B, N, Ns, sr, dil, keep, R):
    n_iters = (keep - 1) * dil + 1

    def body(pos_ref, poss_ref, out_ref, dist_s):
        b = pl.program_id(0)
        p = pos_ref[0]
        q = poss_ref[0]
        p2 = jnp.sum(p * p, axis=1, keepdims=True)
        q2 = jnp.sum(q * q, axis=1)[None, :]
        cross = lax.dot_general(p, q, (((1,), (1,)), ((), ())),
                                preferred_element_type=jnp.float32)
        d2 = p2 + q2 - 2.0 * cross
        dist_s[...] = jnp.sqrt(jnp.maximum(d2, 1e-12))
        iota = lax.broadcasted_iota(jnp.int32, (R, Ns), 1)
        base = b * N

        def step(it, _):
            cdv = dist_s[...]
            m = jnp.min(cdv, axis=1, keepdims=True)
            sel = jnp.min(jnp.where(cdv == m, iota, Ns), axis=1,
                          keepdims=True)

            @pl.when(it % dil == 0)
            def _():
                k = it // dil
                out_ref[0, pl.ds(k, 1), :] = (base + sel[:, 0] * sr)[None, :]

            dist_s[...] = jnp.where(iota == sel, _INF, cdv)
            return 0

        lax.fori_loop(0, n_iters, step, 0)

    return pl.pallas_call(
        body,
        grid=(B, N // R),
        in_specs=[
            pl.BlockSpec((1, R, 3), lambda b, j: (b, j, 0)),
            pl.BlockSpec((1, Ns, 3), lambda b, j: (b, 0, 0)),
        ],
        out_specs=pl.BlockSpec((1, keep, R), lambda b, j: (b, 0, j)),
        out_shape=jax.ShapeDtypeStruct((B, keep, N), jnp.int32),
        scratch_shapes=[pltpu.VMEM((R, Ns), jnp.float32)],
    )


@functools.lru_cache(maxsize=None)
def _make_gather(M, C):
    info = plsc.get_sparse_core_info()
    NW = info.num_cores * info.num_subcores
    CH = 128
    per_w = M // NW
    n_ch = per_w // CH
    mesh = plsc.VectorSubcoreMesh(core_axis_name="c", subcore_axis_name="s")

    @functools.partial(
        pl.kernel,
        out_type=jax.ShapeDtypeStruct((M, C), jnp.float32),
        mesh=mesh,
        scratch_types=[
            pltpu.VMEM((CH,), jnp.int32),
            pltpu.VMEM((CH, C), jnp.float32),
            pltpu.SemaphoreType.DMA,
        ],
    )
    def gk(table_hbm, idx_hbm, out_hbm, idx_v, rows_v, sem):
        wid = lax.axis_index("s") * info.num_cores + lax.axis_index("c")
        base = wid * per_w

        def step(i, _):
            off = base + i * CH
            pltpu.sync_copy(idx_hbm.at[pl.ds(off, CH)], idx_v)
            pltpu.async_copy(table_hbm.at[idx_v], rows_v, sem).wait()
            pltpu.sync_copy(rows_v, out_hbm.at[pl.ds(off, CH)])
            return 0

        lax.fori_loop(0, n_ch, step, 0)

    return gk


def _gather_rows(table, idx_flat):
    return _make_gather(idx_flat.shape[0], table.shape[1])(table, idx_flat)


@functools.lru_cache(maxsize=None)
def _make_stn_pool(B, N, R):
    def body(x_ref, w1, b1, w2, b2, w3, b3, out_ref):
        j = pl.program_id(1)
        h = jnp.maximum(_dot(x_ref[0], w1[...]) + b1[...], 0)
        h = jnp.maximum(_dot(h, w2[...]) + b2[...], 0)
        h = jnp.maximum(_dot(h, w3[...]) + b3[...], 0)
        m = jnp.max(h, axis=0, keepdims=True)[None]

        @pl.when(j == 0)
        def _():
            out_ref[...] = m

        @pl.when(j > 0)
        def _():
            out_ref[...] = jnp.maximum(out_ref[...], m)

    full = lambda *s: pl.BlockSpec(s, lambda b, j: tuple(0 for _ in s))
    return pl.pallas_call(
        body,
        grid=(B, N // R),
        in_specs=[
            pl.BlockSpec((1, R, FDIM), lambda b, j: (b, j, 0)),
            full(FDIM, 64), full(1, 64),
            full(64, 128), full(1, 128),
            full(128, 1024), full(1, 1024),
        ],
        out_specs=pl.BlockSpec((1, 1, 1024), lambda b, j: (b, 0, 0)),
        out_shape=jax.ShapeDtypeStruct((B, 1, 1024), jnp.float32),
    )


@functools.lru_cache(maxsize=None)
def _make_stn_head(B):
    def body(p_ref, w1, b1, w2, b2, w3, b3, out_ref):
        h = jnp.maximum(_dot(p_ref[...], w1[...]) + b1[...], 0)
        h = jnp.maximum(_dot(h, w2[...]) + b2[...], 0)
        out_ref[...] = _dot(h, w3[...]) + b3[...]

    return pl.pallas_call(
        body,
        out_shape=jax.ShapeDtypeStruct((B, FDIM * FDIM), jnp.float32),
    )


@functools.lru_cache(maxsize=None)
def _make_apply_trans(B, N, R):
    def body(x_ref, t_ref, out_ref):
        xt = _dot(x_ref[0], t_ref[0])
        out_ref[0] = jnp.concatenate(
            [xt, jnp.zeros((R, 128 - FDIM), jnp.float32)], axis=1)

    return pl.pallas_call(
        body,
        grid=(B, N // R),
        in_specs=[
            pl.BlockSpec((1, R, FDIM), lambda b, j: (b, j, 0)),
            pl.BlockSpec((1, FDIM, FDIM), lambda b, j: (b, 0, 0)),
        ],
        out_specs=pl.BlockSpec((1, R, 128), lambda b, j: (b, j, 0)),
        out_shape=jax.ShapeDtypeStruct((B, N, 128), jnp.float32),
    )


@functools.lru_cache(maxsize=None)
def _make_edge(B, N, K, P, Cx, Cp, C1, C2):
    def body(x_ref, g_ref, w1, b1, w2, b2, out_ref):
        xc = x_ref[0][:, :Cx]
        xcb = jnp.broadcast_to(xc[:, None, :], (P, K, Cx)).reshape(P * K, Cx)
        gn = g_ref[0].reshape(P * K, Cp)[:, :Cx]
        e = jnp.concatenate([xcb, gn - xcb], axis=1)
        h = _lrelu(_dot(e, w1[...]) + b1[...])
        h = _lrelu(_dot(h, w2[...]) + b2[...])
        out_ref[0] = jnp.max(h.reshape(P, K, C2), axis=1)

    return pl.pallas_call(
        body,
        grid=(B, N // P),
        in_specs=[
            pl.BlockSpec((1, P, Cp), lambda b, j: (b, j, 0)),
            pl.BlockSpec((1, P, K, Cp), lambda b, j: (b, j, 0, 0)),
            pl.BlockSpec((2 * Cx, C1), lambda b, j: (0, 0)),
            pl.BlockSpec((1, C1), lambda b, j: (0, 0)),
            pl.BlockSpec((C1, C2), lambda b, j: (0, 0)),
            pl.BlockSpec((1, C2), lambda b, j: (0, 0)),
        ],
        out_specs=pl.BlockSpec((1, P, C2), lambda b, j: (b, j, 0)),
        out_shape=jax.ShapeDtypeStruct((B, N, C2), jnp.float32),
    )


@functools.lru_cache(maxsize=None)
def _make_tail(B, N, P, NC):
    def body(x_ref, attw, attb, r1w1, r1b1, r1w2, r1b2,
             r2w1, r2b1, r2w2, r2b2, r2wr, r2br, ow, ob, out_ref):
        xv = x_ref[0]
        logit = jnp.sum(xv * attw[...], axis=1, keepdims=True) + attb[...]
        xv = xv * (1.0 / (1.0 + jnp.exp(-logit)))
        h = _lrelu(_dot(xv, r1w1[...]) + r1b1[...])
        h = _lrelu(_dot(h, r1w2[...]) + r1b2[...])
        xv = h + xv
        h = _lrelu(_dot(xv, r2w1[...]) + r2b1[...])
        h = _lrelu(_dot(h, r2w2[...]) + r2b2[...])
        xv = h + _dot(xv, r2wr[...]) + r2br[...]
        out_ref[0] = _dot(xv, ow[...]) + ob[...]

    full = lambda *s: pl.BlockSpec(s, lambda b, j: tuple(0 for _ in s))
    return pl.pallas_call(
        body,
        grid=(B, N // P),
        in_specs=[
            pl.BlockSpec((1, P, 512), lambda b, j: (b, j, 0)),
            full(1, 512), full(1, 1),
            full(512, 512), full(1, 512), full(512, 512), full(1, 512),
            full(512, 256), full(1, 256), full(256, 256), full(1, 256),
            full(512, 256), full(1, 256),
            full(256, NC), full(1, NC),
        ],
        out_specs=pl.BlockSpec((1, P, NC), lambda b, j: (b, j, 0)),
        out_shape=jax.ShapeDtypeStruct((B, N, NC), jnp.float32),
    )


def _rb(b):
    return b.reshape(1, -1)


def kernel(x, pos, params):
    B, N, D = x.shape
    K = KNN
    R = 256
    P = 256

    idx_local = _make_knn(B, N, N, 1, 1, K, 1024)(pos, pos)
    sample_idx = []
    for sr in (4, 8, 16):
        pos_s = pos[:, ::sr, :]
        Ns = pos_s.shape[1]
        sample_idx.append(
            _make_knn(B, N, Ns, sr, 8, K, 4096)(pos, pos_s))

    def flat_idx(ix):
        return ix.transpose(0, 2, 1).reshape(-1)

    ps = params["stn"]
    pooled = _make_stn_pool(B, N, R)(
        x, ps["conv1"]["W"], _rb(ps["conv1"]["b"]),
        ps["conv2"]["W"], _rb(ps["conv2"]["b"]),
        ps["conv3"]["W"], _rb(ps["conv3"]["b"]))
    t = _make_stn_head(B)(
        pooled.reshape(B, 1024), ps["fc1"]["W"], _rb(ps["fc1"]["b"]),
        ps["fc2"]["W"], _rb(ps["fc2"]["b"]),
        ps["fc3"]["W"], _rb(ps["fc3"]["b"]))
    trans = t.reshape(B, FDIM, FDIM)

    pe = params["e_local"]
    xtp = _make_apply_trans(B, N, R)(x, trans)
    g = _gather_rows(xtp.reshape(B * N, 128), flat_idx(idx_local))
    xcur = _make_edge(B, N, K, P, FDIM, 128, 128, 256)(
        xtp, g.reshape(B, N, K, 128),
        pe["l1"]["W"], _rb(pe["l1"]["b"]), pe["l2"]["W"], _rb(pe["l2"]["b"]))

    for i, name in enumerate(("e0", "e1", "e2")):
        pe = params[name]
        Cx = pe["l1"]["W"].shape[0] // 2
        C1 = pe["l1"]["W"].shape[1]
        C2 = pe["l2"]["W"].shape[1]
        g = _gather_rows(xcur.reshape(B * N, Cx), flat_idx(sample_idx[i]))
        xcur = _make_edge(B, N, K, P, Cx, Cx, C1, C2)(
            xcur, g.reshape(B, N, K, Cx),
            pe["l1"]["W"], _rb(pe["l1"]["b"]), pe["l2"]["W"], _rb(pe["l2"]["b"]))

    pa, p1, p2, po = (params["attention"], params["res1"], params["res2"],
                      params["out"])
    NC = po["W"].shape[1]
    out = _make_tail(B, N, P, NC)(
        xcur,
        pa["att"]["W"].reshape(1, 512), pa["att"]["b"].reshape(1, 1),
        p1["l1"]["W"], _rb(p1["l1"]["b"]), p1["l2"]["W"], _rb(p1["l2"]["b"]),
        p2["l1"]["W"], _rb(p2["l1"]["b"]), p2["l2"]["W"], _rb(p2["l2"]["b"]),
        p2["rescale"]["W"], _rb(p2["rescale"]["b"]),
        po["W"], _rb(po["b"]))
    return out


# speedup vs baseline: 1.5144x; 1.0104x over previous
import functools

import jax
import jax.numpy as jnp
from jax import lax
from jax.experimental import pallas as pl
from jax.experimental.pallas import tpu as pltpu
from jax.experimental.pallas import tpu_sc as plsc

FDIM = 24
KNN = 16
_INF = float("inf")


def _lrelu(v):
    return jnp.where(v >= 0, v, 0.2 * v)


def _dot(a, b):
    return jnp.dot(a, b, preferred_element_type=jnp.float32)


@functools.lru_cache(maxsize=None)
def _make_knn(B, N, Ns, sr, dil, keep, R):
    n_iters = (keep - 1) * dil + 1

    def body(pos_ref, poss_ref, out_ref, dist_s):
        b = pl.program_id(0)
        p = pos_ref[0]
        q = poss_ref[0]
        p2 = jnp.sum(p * p, axis=1, keepdims=True)
        q2 = jnp.sum(q * q, axis=1)[None, :]
        cross = lax.dot_general(p, q, (((1,), (1,)), ((), ())),
                                preferred_element_type=jnp.float32)
        d2 = p2 + q2 - 2.0 * cross
        dist_s[...] = jnp.sqrt(jnp.maximum(d2, 1e-12))
        iota = lax.broadcasted_iota(jnp.int32, (R, Ns), 1)
        base = b * N

        def step(it, sp):
            masked = jnp.where(iota == sp, _INF, dist_s[...])
            dist_s[...] = masked
            m = jnp.min(masked, axis=1, keepdims=True)
            sel = jnp.min(jnp.where(masked == m, iota, Ns), axis=1,
                          keepdims=True)

            @pl.when(it % dil == 0)
            def _():
                k = it // dil
                out_ref[0, pl.ds(k, 1), :] = (base + sel[:, 0] * sr)[None, :]

            return sel

        lax.fori_loop(0, n_iters, step, jnp.full((R, 1), -1, jnp.int32))

    return pl.pallas_call(
        body,
        grid=(B, N // R),
        in_specs=[
            pl.BlockSpec((1, R, 3), lambda b, j: (b, j, 0)),
            pl.BlockSpec((1, Ns, 3), lambda b, j: (b, 0, 0)),
        ],
        out_specs=pl.BlockSpec((1, keep, R), lambda b, j: (b, 0, j)),
        out_shape=jax.ShapeDtypeStruct((B, keep, N), jnp.int32),
        scratch_shapes=[pltpu.VMEM((R, Ns), jnp.float32)],
    )


@functools.lru_cache(maxsize=None)
def _make_gather(M, C):
    info = plsc.get_sparse_core_info()
    NW = info.num_cores * info.num_subcores
    CH = 128
    per_w = M // NW
    n_ch = per_w // CH
    mesh = plsc.VectorSubcoreMesh(core_axis_name="c", subcore_axis_name="s")

    @functools.partial(
        pl.kernel,
        out_type=jax.ShapeDtypeStruct((M, C), jnp.float32),
        mesh=mesh,
        scratch_types=[
            pltpu.VMEM((CH,), jnp.int32),
            pltpu.VMEM((CH, C), jnp.float32),
            pltpu.SemaphoreType.DMA,
        ],
    )
    def gk(table_hbm, idx_hbm, out_hbm, idx_v, rows_v, sem):
        wid = lax.axis_index("s") * info.num_cores + lax.axis_index("c")
        base = wid * per_w

        def step(i, _):
            off = base + i * CH
            pltpu.sync_copy(idx_hbm.at[pl.ds(off, CH)], idx_v)
            pltpu.async_copy(table_hbm.at[idx_v], rows_v, sem).wait()
            pltpu.sync_copy(rows_v, out_hbm.at[pl.ds(off, CH)])
            return 0

        lax.fori_loop(0, n_ch, step, 0)

    return gk


def _gather_rows(table, idx_flat):
    return _make_gather(idx_flat.shape[0], table.shape[1])(table, idx_flat)


@functools.lru_cache(maxsize=None)
def _make_stn_pool(B, N, R):
    def body(x_ref, w1, b1, w2, b2, w3, b3, out_ref):
        j = pl.program_id(1)
        h = jnp.maximum(_dot(x_ref[0], w1[...]) + b1[...], 0)
        h = jnp.maximum(_dot(h, w2[...]) + b2[...], 0)
        h = jnp.maximum(_dot(h, w3[...]) + b3[...], 0)
        m = jnp.max(h, axis=0, keepdims=True)[None]

        @pl.when(j == 0)
        def _():
            out_ref[...] = m

        @pl.when(j > 0)
        def _():
            out_ref[...] = jnp.maximum(out_ref[...], m)

    full = lambda *s: pl.BlockSpec(s, lambda b, j: tuple(0 for _ in s))
    return pl.pallas_call(
        body,
        grid=(B, N // R),
        in_specs=[
            pl.BlockSpec((1, R, FDIM), lambda b, j: (b, j, 0)),
            full(FDIM, 64), full(1, 64),
            full(64, 128), full(1, 128),
            full(128, 1024), full(1, 1024),
        ],
        out_specs=pl.BlockSpec((1, 1, 1024), lambda b, j: (b, 0, 0)),
        out_shape=jax.ShapeDtypeStruct((B, 1, 1024), jnp.float32),
    )


@functools.lru_cache(maxsize=None)
def _make_stn_head(B):
    def body(p_ref, w1, b1, w2, b2, w3, b3, out_ref):
        h = jnp.maximum(_dot(p_ref[...], w1[...]) + b1[...], 0)
        h = jnp.maximum(_dot(h, w2[...]) + b2[...], 0)
        out_ref[...] = _dot(h, w3[...]) + b3[...]

    return pl.pallas_call(
        body,
        out_shape=jax.ShapeDtypeStruct((B, FDIM * FDIM), jnp.float32),
    )


@functools.lru_cache(maxsize=None)
def _make_apply_trans(B, N, R):
    def body(x_ref, t_ref, out_ref):
        xt = _dot(x_ref[0], t_ref[0])
        out_ref[0] = jnp.concatenate(
            [xt, jnp.zeros((R, 128 - FDIM), jnp.float32)], axis=1)

    return pl.pallas_call(
        body,
        grid=(B, N // R),
        in_specs=[
            pl.BlockSpec((1, R, FDIM), lambda b, j: (b, j, 0)),
            pl.BlockSpec((1, FDIM, FDIM), lambda b, j: (b, 0, 0)),
        ],
        out_specs=pl.BlockSpec((1, R, 128), lambda b, j: (b, j, 0)),
        out_shape=jax.ShapeDtypeStruct((B, N, 128), jnp.float32),
    )


@functools.lru_cache(maxsize=None)
def _make_edge(B, N, K, P, Cx, Cp, C1, C2):
    def body(x_ref, g_ref, w1, b1, w2, b2, out_ref):
        xc = x_ref[0][:, :Cx]
        xcb = jnp.broadcast_to(xc[:, None, :], (P, K, Cx)).reshape(P * K, Cx)
        gn = g_ref[0].reshape(P * K, Cp)[:, :Cx]
        e = jnp.concatenate([xcb, gn - xcb], axis=1)
        h = _lrelu(_dot(e, w1[...]) + b1[...])
        h = _lrelu(_dot(h, w2[...]) + b2[...])
        out_ref[0] = jnp.max(h.reshape(P, K, C2), axis=1)

    return pl.pallas_call(
        body,
        grid=(B, N // P),
        in_specs=[
            pl.BlockSpec((1, P, Cp), lambda b, j: (b, j, 0)),
            pl.BlockSpec((1, P, K, Cp), lambda b, j: (b, j, 0, 0)),
            pl.BlockSpec((2 * Cx, C1), lambda b, j: (0, 0)),
            pl.BlockSpec((1, C1), lambda b, j: (0, 0)),
            pl.BlockSpec((C1, C2), lambda b, j: (0, 0)),
            pl.BlockSpec((1, C2), lambda b, j: (0, 0)),
        ],
        out_specs=pl.BlockSpec((1, P, C2), lambda b, j: (b, j, 0)),
        out_shape=jax.ShapeDtypeStruct((B, N, C2), jnp.float32),
    )


@functools.lru_cache(maxsize=None)
def _make_tail(B, N, P, NC):
    def body(x_ref, attw, attb, r1w1, r1b1, r1w2, r1b2,
             r2w1, r2b1, r2w2, r2b2, r2wr, r2br, ow, ob, out_ref):
        xv = x_ref[0]
        logit = jnp.sum(xv * attw[...], axis=1, keepdims=True) + attb[...]
        xv = xv * (1.0 / (1.0 + jnp.exp(-logit)))
        h = _lrelu(_dot(xv, r1w1[...]) + r1b1[...])
        h = _lrelu(_dot(h, r1w2[...]) + r1b2[...])
        xv = h + xv
        h = _lrelu(_dot(xv, r2w1[...]) + r2b1[...])
        h = _lrelu(_dot(h, r2w2[...]) + r2b2[...])
        xv = h + _dot(xv, r2wr[...]) + r2br[...]
        out_ref[0] = _dot(xv, ow[...]) + ob[...]

    full = lambda *s: pl.BlockSpec(s, lambda b, j: tuple(0 for _ in s))
    return pl.pallas_call(
        body,
        grid=(B, N // P),
        in_specs=[
            pl.BlockSpec((1, P, 512), lambda b, j: (b, j, 0)),
            full(1, 512), full(1, 1),
            full(512, 512), full(1, 512), full(512, 512), full(1, 512),
            full(512, 256), full(1, 256), full(256, 256), full(1, 256),
            full(512, 256), full(1, 256),
            full(256, NC), full(1, NC),
        ],
        out_specs=pl.BlockSpec((1, P, NC), lambda b, j: (b, j, 0)),
        out_shape=jax.ShapeDtypeStruct((B, N, NC), jnp.float32),
    )


def _rb(b):
    return b.reshape(1, -1)


def kernel(x, pos, params):
    B, N, D = x.shape
    K = KNN
    R = 256
    P = 256

    idx_local = _make_knn(B, N, N, 1, 1, K, 1024)(pos, pos)
    sample_idx = []
    for sr in (4, 8, 16):
        pos_s = pos[:, ::sr, :]
        Ns = pos_s.shape[1]
        sample_idx.append(
            _make_knn(B, N, Ns, sr, 8, K, 4096)(pos, pos_s))

    def flat_idx(ix):
        return ix.transpose(0, 2, 1).reshape(-1)

    ps = params["stn"]
    pooled = _make_stn_pool(B, N, R)(
        x, ps["conv1"]["W"], _rb(ps["conv1"]["b"]),
        ps["conv2"]["W"], _rb(ps["conv2"]["b"]),
        ps["conv3"]["W"], _rb(ps["conv3"]["b"]))
    t = _make_stn_head(B)(
        pooled.reshape(B, 1024), ps["fc1"]["W"], _rb(ps["fc1"]["b"]),
        ps["fc2"]["W"], _rb(ps["fc2"]["b"]),
        ps["fc3"]["W"], _rb(ps["fc3"]["b"]))
    trans = t.reshape(B, FDIM, FDIM)

    pe = params["e_local"]
    xtp = _make_apply_trans(B, N, R)(x, trans)
    g = _gather_rows(xtp.reshape(B * N, 128), flat_idx(idx_local))
    xcur = _make_edge(B, N, K, P, FDIM, 128, 128, 256)(
        xtp, g.reshape(B, N, K, 128),
        pe["l1"]["W"], _rb(pe["l1"]["b"]), pe["l2"]["W"], _rb(pe["l2"]["b"]))

    for i, name in enumerate(("e0", "e1", "e2")):
        pe = params[name]
        Cx = pe["l1"]["W"].shape[0] // 2
        C1 = pe["l1"]["W"].shape[1]
        C2 = pe["l2"]["W"].shape[1]
        g = _gather_rows(xcur.reshape(B * N, Cx), flat_idx(sample_idx[i]))
        xcur = _make_edge(B, N, K, P, Cx, Cx, C1, C2)(
            xcur, g.reshape(B, N, K, Cx),
            pe["l1"]["W"], _rb(pe["l1"]["b"]), pe["l2"]["W"], _rb(pe["l2"]["b"]))

    pa, p1, p2, po = (params["attention"], params["res1"], params["res2"],
                      params["out"])
    NC = po["W"].shape[1]
    out = _make_tail(B, N, P, NC)(
        xcur,
        pa["att"]["W"].reshape(1, 512), pa["att"]["b"].reshape(1, 1),
        p1["l1"]["W"], _rb(p1["l1"]["b"]), p1["l2"]["W"], _rb(p1["l2"]["b"]),
        p2["l1"]["W"], _rb(p2["l1"]["b"]), p2["l2"]["W"], _rb(p2["l2"]["b"]),
        p2["rescale"]["W"], _rb(p2["rescale"]["b"]),
        po["W"], _rb(po["b"]))
    return out
